# baseline (speedup 1.0000x reference)
import numpy as np
import jax
import jax.numpy as jnp
from jax.experimental import pallas as pl
from jax.experimental.pallas import tpu as pltpu

_SLOW_FRAMES = 8


def _pack_body(info_ref, frames_ref, slow_ref, fast_ref):
    t = pl.program_id(0)
    x = frames_ref[...]
    fast_ref[...] = x

    @pl.when(info_ref[1, t] == 1)
    def _():
        slow_ref[...] = x


def kernel(frames):
    C, T, H, W = frames.shape
    idx = np.linspace(0.0, float(T - 1), _SLOW_FRAMES).astype(np.int32)
    sel = np.zeros((T,), np.int32)
    sel[idx] = 1
    prev_slot = np.maximum(np.searchsorted(idx, np.arange(T), side="right") - 1, 0)
    info = jnp.asarray(np.stack([prev_slot, sel]).astype(np.int32))

    grid_spec = pltpu.PrefetchScalarGridSpec(
        num_scalar_prefetch=1,
        grid=(T,),
        in_specs=[
            pl.BlockSpec((C, 1, H, W), lambda t, info: (0, t, 0, 0)),
        ],
        out_specs=[
            pl.BlockSpec((C, 1, H, W), lambda t, info: (0, info[0, t], 0, 0)),
            pl.BlockSpec((C, 1, H, W), lambda t, info: (0, t, 0, 0)),
        ],
    )
    slow, fast = pl.pallas_call(
        _pack_body,
        grid_spec=grid_spec,
        out_shape=(
            jax.ShapeDtypeStruct((C, _SLOW_FRAMES, H, W), frames.dtype),
            jax.ShapeDtypeStruct((C, T, H, W), frames.dtype),
        ),
    )(info, frames)
    return (slow, fast)


# speedup vs baseline: 1.2353x; 1.2353x over previous
import numpy as np
import jax
import jax.numpy as jnp
from jax.experimental import pallas as pl
from jax.experimental.pallas import tpu as pltpu

_SLOW_FRAMES = 8


def _gather_body(idx_ref, frames_ref, slow_ref):
    slow_ref[...] = frames_ref[...]


def kernel(frames):
    C, T, H, W = frames.shape
    idx = jnp.asarray(
        np.linspace(0.0, float(T - 1), _SLOW_FRAMES).astype(np.int32)
    )

    grid_spec = pltpu.PrefetchScalarGridSpec(
        num_scalar_prefetch=1,
        grid=(_SLOW_FRAMES,),
        in_specs=[
            pl.BlockSpec((C, 1, H, W), lambda j, idx: (0, idx[j], 0, 0)),
        ],
        out_specs=pl.BlockSpec((C, 1, H, W), lambda j, idx: (0, j, 0, 0)),
    )
    slow = pl.pallas_call(
        _gather_body,
        grid_spec=grid_spec,
        out_shape=jax.ShapeDtypeStruct((C, _SLOW_FRAMES, H, W), frames.dtype),
    )(idx, frames)
    return (slow, frames)
